# baseline (speedup 1.0000x reference)
import functools
import math

import jax
import jax.numpy as jnp
import numpy as np
from jax.experimental import pallas as pl
from jax.experimental.pallas import tpu as pltpu

_SIZES = [32, 64, 128, 256, 512]
_RATIOS = [0.5, 1.0, 2.0]

_LEVELS = [(64, 64), (32, 32), (16, 16)]


@functools.lru_cache(maxsize=None)
def _anchors_const(img_h, grids):
    per_all = []
    for grid in grids:
        scale = img_h / grid
        steps = (np.arange(grid, dtype=np.float32)
                 * np.float32(scale)).astype(np.float32)
        x, y = np.meshgrid(steps, steps, indexing='ij')
        for s in _SIZES:
            for r in _RATIOS:
                rs = math.sqrt(r)
                aw = np.full((grid, grid), np.float32(s * rs), dtype=np.float32)
                ah = np.full((grid, grid), np.float32(s / rs), dtype=np.float32)
                a = np.stack((x, y, aw, ah)).transpose(1, 2, 0).reshape(-1, 4)
                per_all.append(a)
    return np.concatenate(per_all, axis=0)


def _rpn_head_kernel(x3, x4, x5, w2, cb, hw_, hb, o3, o4, o5, xs, xim):
    for (h, w), x, o in zip(_LEVELS, (x3, x4, x5), (o3, o4, o5)):
        hw = h * w
        base = w + 1
        xs[:, 0:base] = jnp.zeros((256, base), dtype=jnp.bfloat16)
        xs[:, base + hw:base + hw + base] = jnp.zeros((256, base),
                                                      dtype=jnp.bfloat16)
        xs[:, base:base + hw] = x[...].astype(jnp.bfloat16)

        col = jax.lax.broadcasted_iota(jnp.int32, (1, hw), 1) % w
        for k, (dy, dx) in enumerate(
                (dy, dx) for dy in (-1, 0, 1) for dx in (-1, 0, 1)):
            sl = xs[:, base + dy * w + dx:base + dy * w + dx + hw]
            if dx == -1:
                sl = jnp.where(col != 0, sl, jnp.bfloat16(0))
            elif dx == 1:
                sl = jnp.where(col != w - 1, sl, jnp.bfloat16(0))
            xim[k * 256:(k + 1) * 256, 0:hw] = sl

        acc = jnp.dot(w2[...], xim[:, 0:hw],
                      preferred_element_type=jnp.float32)
        t = jnp.maximum(acc + cb[...], 0.0).astype(jnp.bfloat16)
        o[...] = jnp.dot(hw_[...], t,
                         preferred_element_type=jnp.float32) + hb[...]


def kernel(images, feat_p3, feat_p4, feat_p5, conv_w, conv_b,
           cls_w, cls_b, bbox_w, bbox_b):
    feats = (feat_p3, feat_p4, feat_p5)
    xs_in = [f.reshape(256, h * w) for f, (h, w) in zip(feats, _LEVELS)]

    w2 = conv_w.transpose(0, 2, 3, 1).reshape(256, 2304).astype(jnp.bfloat16)
    cb = conv_b.reshape(256, 1)
    head_w = jnp.concatenate(
        [cls_w.reshape(15, 256), bbox_w.reshape(60, 256)]).astype(jnp.bfloat16)
    head_b = jnp.concatenate([cls_b, bbox_b]).reshape(75, 1)

    out_shapes = tuple(jax.ShapeDtypeStruct((75, h * w), jnp.float32)
                       for h, w in _LEVELS)
    hmax, wmax = _LEVELS[0]

    o3, o4, o5 = pl.pallas_call(
        _rpn_head_kernel,
        out_shape=out_shapes,
        scratch_shapes=[
            pltpu.VMEM((256, hmax * wmax + 2 * wmax + 2), jnp.bfloat16),
            pltpu.VMEM((2304, hmax * wmax), jnp.bfloat16),
        ],
    )(xs_in[0], xs_in[1], xs_in[2], w2, cb, head_w, head_b)

    flat = jnp.concatenate(
        [o.reshape(1, -1) for o in (o3, o4, o5)], axis=1)

    anchors = jnp.asarray(
        _anchors_const(images.shape[-2], tuple(h for h, _ in _LEVELS)))
    return (flat, anchors)
